# Initial kernel scaffold; baseline (speedup 1.0000x reference)
import functools

import jax
import jax.numpy as jnp
from jax import lax
from jax.experimental import pallas as pl
from jax.experimental.pallas import tpu as pltpu
from jax.experimental.pallas import tpu_sc as plsc

N = 10000
E = 320000
D = 128
DH = 64

NC = 2
NS = 16
NW = NC * NS
EPW = E // NW
CHUNK = 80
NCHUNK = EPW // CHUNK
RPT = N // NS

BN_EPS = 1e-5
AGG_EPS = 1e-6


def _node_mm_body(h_ref, wa_ref, wb_ref, wd_ref, we_ref,
                  ah_ref, b0_ref, b1_ref, d0_ref, d1_ref, e0_ref, e1_ref):
    h = h_ref[...]
    dn = (((1,), (1,)), ((), ()))
    ah_ref[...] = lax.dot_general(h, wa_ref[...], dn,
                                  preferred_element_type=jnp.float32)
    bh = lax.dot_general(h, wb_ref[...], dn, preferred_element_type=jnp.float32)
    dh = lax.dot_general(h, wd_ref[...], dn, preferred_element_type=jnp.float32)
    eh = lax.dot_general(h, we_ref[...], dn, preferred_element_type=jnp.float32)
    b0_ref[...] = bh[:, :DH]
    b1_ref[...] = bh[:, DH:]
    d0_ref[...] = dh[:, :DH]
    d1_ref[...] = dh[:, DH:]
    e0_ref[...] = eh[:, :DH]
    e1_ref[...] = eh[:, DH:]


def _node_mm(h, WA, WB, WD, WE):
    BN = 1000
    grid = (N // BN,)
    full = pl.BlockSpec((BN, D), lambda i: (i, 0))
    half = pl.BlockSpec((BN, DH), lambda i: (i, 0))
    w = pl.BlockSpec((D, D), lambda i: (0, 0))
    return pl.pallas_call(
        _node_mm_body,
        grid=grid,
        in_specs=[full, w, w, w, w],
        out_specs=[full, half, half, half, half, half, half],
        out_shape=[jax.ShapeDtypeStruct((N, D), jnp.float32)]
        + [jax.ShapeDtypeStruct((N, DH), jnp.float32)] * 6,
    )(h, WA, WB, WD, WE)


def _edge_mm_body(e_ref, wc_ref, c0_ref, c1_ref):
    dn = (((1,), (1,)), ((), ()))
    ce = lax.dot_general(e_ref[...], wc_ref[...], dn,
                         preferred_element_type=jnp.float32)
    c0_ref[...] = ce[:, :DH]
    c1_ref[...] = ce[:, DH:]


def _edge_mm(e, WC):
    BE = 2000
    grid = (E // BE,)
    return pl.pallas_call(
        _edge_mm_body,
        grid=grid,
        in_specs=[pl.BlockSpec((BE, D), lambda i: (i, 0)),
                  pl.BlockSpec((D, D), lambda i: (0, 0))],
        out_specs=[pl.BlockSpec((BE, DH), lambda i: (i, 0)),
                   pl.BlockSpec((BE, DH), lambda i: (i, 0))],
        out_shape=[jax.ShapeDtypeStruct((E, DH), jnp.float32)] * 2,
    )(e, WC)


def _sc_edge_body(src_hbm, dst_hbm, ce_hbm, bh_hbm, dh_hbm, eh_hbm, zeros_hbm,
                  enew_hbm, acc_hbm,
                  idx_s, idx_d, ce_v, dh_v, eh_v, bh_v, nd_v, acc_sh, sem):
    c = lax.axis_index("c")
    s = lax.axis_index("s")
    wid = s * NC + c
    ebase = wid * EPW

    pltpu.sync_copy(zeros_hbm.at[pl.ds(s * RPT, RPT)],
                    acc_sh.at[pl.ds(s * RPT, RPT)])
    plsc.subcore_barrier()

    def chunk_body(ci, carry):
        base = ebase + ci * CHUNK
        pltpu.sync_copy(src_hbm.at[pl.ds(base, CHUNK)], idx_s)
        pltpu.sync_copy(dst_hbm.at[pl.ds(base, CHUNK)], idx_d)
        pltpu.async_copy(bh_hbm.at[idx_s], bh_v, sem).wait()
        pltpu.async_copy(dh_hbm.at[idx_s], dh_v, sem).wait()
        pltpu.async_copy(eh_hbm.at[idx_d], eh_v, sem).wait()
        pltpu.sync_copy(ce_hbm.at[pl.ds(base, CHUNK)], ce_v)

        def edge_body(i, carry2):
            for j in range(DH // 16):
                sl = pl.ds(j * 16, 16)
                en = ce_v[i, sl] + dh_v[i, sl] + eh_v[i, sl]
                ce_v[i, sl] = en
                sig = 1.0 / (1.0 + jnp.exp(-en))
                nd_v[i, pl.ds(DH + j * 16, 16)] = sig
                nd_v[i, pl.ds(j * 16, 16)] = sig * bh_v[i, sl]
            return carry2

        lax.fori_loop(0, CHUNK, edge_body, 0)
        pltpu.sync_copy(ce_v, enew_hbm.at[pl.ds(base, CHUNK)])
        pltpu.sync_copy(nd_v, acc_sh.at[idx_d], add=True)
        return carry

    lax.fori_loop(0, NCHUNK, chunk_body, 0)
    plsc.subcore_barrier()
    pltpu.sync_copy(acc_sh.at[pl.ds(s * RPT, RPT)],
                    acc_hbm.at[c, pl.ds(s * RPT, RPT)])


_sc_edge_pass = pl.kernel(
    _sc_edge_body,
    out_type=[jax.ShapeDtypeStruct((E, DH), jnp.float32),
              jax.ShapeDtypeStruct((NC, N, 2 * DH), jnp.float32)],
    mesh=plsc.VectorSubcoreMesh(core_axis_name="c", subcore_axis_name="s"),
    scratch_types=[
        pltpu.VMEM((CHUNK,), jnp.int32),
        pltpu.VMEM((CHUNK,), jnp.int32),
        pltpu.VMEM((CHUNK, DH), jnp.float32),
        pltpu.VMEM((CHUNK, DH), jnp.float32),
        pltpu.VMEM((CHUNK, DH), jnp.float32),
        pltpu.VMEM((CHUNK, DH), jnp.float32),
        pltpu.VMEM((CHUNK, 2 * DH), jnp.float32),
        pltpu.VMEM_SHARED((N, 2 * DH), jnp.float32),
        pltpu.SemaphoreType.DMA,
    ],
)


def _node_fin_body(h_ref, ah_ref, acc0_ref, acc1_ref, sn_ref, g_ref, b_ref,
                   out_ref, stat_ref):
    j = pl.program_id(0)
    a0 = acc0_ref[0] + acc0_ref[1]
    a1 = acc1_ref[0] + acc1_ref[1]
    num = jnp.concatenate([a0[:, :DH], a1[:, :DH]], axis=1)
    den = jnp.concatenate([a0[:, DH:], a1[:, DH:]], axis=1)
    hn = (ah_ref[...] + num / (den + AGG_EPS)) * sn_ref[...]

    @pl.when(jnp.logical_and(j == 0, pl.program_id(1) == 0))
    def _():
        stat_ref[...] = jnp.zeros_like(stat_ref)

    @pl.when(j == 0)
    def _():
        stat_ref[0:1, :] += jnp.sum(hn, axis=0, keepdims=True)
        stat_ref[1:2, :] += jnp.sum(hn * hn, axis=0, keepdims=True)

    @pl.when(j == 1)
    def _():
        m = stat_ref[0:1, :] / N
        v = stat_ref[1:2, :] / N - m * m
        scale = g_ref[...] / jnp.sqrt(v + BN_EPS)
        out_ref[...] = h_ref[...] + (hn - m) * scale + b_ref[...]


def _node_fin(h, ah, acc0, acc1, snorm_n, gamma_h, beta_h):
    BN = 2000
    grid = (2, N // BN)
    full = pl.BlockSpec((BN, D), lambda j, i: (i, 0))
    accs = pl.BlockSpec((NC, BN, D), lambda j, i: (0, i, 0))
    vec = pl.BlockSpec((1, D), lambda j, i: (0, 0))
    return pl.pallas_call(
        _node_fin_body,
        grid=grid,
        in_specs=[full, full, accs, accs,
                  pl.BlockSpec((BN, 1), lambda j, i: (i, 0)), vec, vec],
        out_specs=full,
        out_shape=jax.ShapeDtypeStruct((N, D), jnp.float32),
        scratch_shapes=[pltpu.VMEM((8, D), jnp.float32)],
    )(h, ah, acc0, acc1, snorm_n, gamma_h, beta_h)


def _edge_fin_body(e_ref, en0_ref, en1_ref, sn_ref, g_ref, b_ref,
                   out_ref, stat_ref):
    j = pl.program_id(0)
    y = jnp.concatenate([en0_ref[...], en1_ref[...]], axis=1) * sn_ref[...]

    @pl.when(jnp.logical_and(j == 0, pl.program_id(1) == 0))
    def _():
        stat_ref[...] = jnp.zeros_like(stat_ref)

    @pl.when(j == 0)
    def _():
        stat_ref[0:1, :] += jnp.sum(y, axis=0, keepdims=True)
        stat_ref[1:2, :] += jnp.sum(y * y, axis=0, keepdims=True)

    @pl.when(j == 1)
    def _():
        m = stat_ref[0:1, :] / E
        v = stat_ref[1:2, :] / E - m * m
        scale = g_ref[...] / jnp.sqrt(v + BN_EPS)
        out_ref[...] = e_ref[...] + (y - m) * scale + b_ref[...]


def _edge_fin(e, en0, en1, snorm_e, gamma_e, beta_e):
    BE = 2000
    grid = (2, E // BE)
    half = pl.BlockSpec((BE, DH), lambda j, i: (i, 0))
    vec = pl.BlockSpec((1, D), lambda j, i: (0, 0))
    lazy = pl.BlockSpec((BE, D), lambda j, i: (jnp.where(j == 1, i, 0), 0))
    return pl.pallas_call(
        _edge_fin_body,
        grid=grid,
        in_specs=[lazy, half, half,
                  pl.BlockSpec((BE, 1), lambda j, i: (i, 0)), vec, vec],
        out_specs=lazy,
        out_shape=jax.ShapeDtypeStruct((E, D), jnp.float32),
        scratch_shapes=[pltpu.VMEM((8, D), jnp.float32)],
    )(e, en0, en1, snorm_e, gamma_e, beta_e)


def kernel(h, e, edge_index, snorm_n, snorm_e, WA, WB, WC, WD, WE,
           gamma_h, beta_h, gamma_e, beta_e):
    src = edge_index[0]
    dst = edge_index[1]
    ah, b0, b1, d0, d1, e0t, e1t = _node_mm(h, WA, WB, WD, WE)
    ce0, ce1 = _edge_mm(e, WC)
    zeros = jnp.zeros((N, 2 * DH), jnp.float32)
    en0, acc0 = _sc_edge_pass(src, dst, ce0, b0, d0, e0t, zeros)
    en1, acc1 = _sc_edge_pass(src, dst, ce1, b1, d1, e1t, zeros)
    h_out = _node_fin(h, ah, acc0, acc1, snorm_n,
                      gamma_h.reshape(1, D), beta_h.reshape(1, D))
    e_out = _edge_fin(e, en0, en1, snorm_e,
                      gamma_e.reshape(1, D), beta_e.reshape(1, D))
    return (h_out, e_out)


# speedup vs baseline: 1.9423x; 1.9423x over previous
import functools

import jax
import jax.numpy as jnp
from jax import lax
from jax.experimental import pallas as pl
from jax.experimental.pallas import tpu as pltpu
from jax.experimental.pallas import tpu_sc as plsc

N = 10000
E = 320000
D = 128
DH = 64

NC = 2
NS = 16
NW = NC * NS
EPW = E // NW
CHUNK = 80
NCHUNK = EPW // CHUNK
NPAD = 10240
RPT = NPAD // NS

BN_EPS = 1e-5
AGG_EPS = 1e-6


def _node_mm_body(h_ref, wa_ref, wb_ref, wd_ref, we_ref,
                  ah_ref, db0_ref, db1_ref, ehf_ref):
    h = h_ref[...]
    dn = (((1,), (1,)), ((), ()))
    ah_ref[...] = lax.dot_general(h, wa_ref[...], dn,
                                  preferred_element_type=jnp.float32)
    bh = lax.dot_general(h, wb_ref[...], dn, preferred_element_type=jnp.float32)
    dh = lax.dot_general(h, wd_ref[...], dn, preferred_element_type=jnp.float32)
    ehf_ref[...] = lax.dot_general(h, we_ref[...], dn,
                                   preferred_element_type=jnp.float32)
    db0_ref[...] = jnp.concatenate([dh[:, :DH], bh[:, :DH]], axis=1)
    db1_ref[...] = jnp.concatenate([dh[:, DH:], bh[:, DH:]], axis=1)


def _node_mm(h, WA, WB, WD, WE):
    BN = 1000
    grid = (N // BN,)
    full = pl.BlockSpec((BN, D), lambda i: (i, 0))
    w = pl.BlockSpec((D, D), lambda i: (0, 0))
    return pl.pallas_call(
        _node_mm_body,
        grid=grid,
        in_specs=[full, w, w, w, w],
        out_specs=[full, full, full, full],
        out_shape=[jax.ShapeDtypeStruct((N, D), jnp.float32)] * 4,
    )(h, WA, WB, WD, WE)


def _edge_mm_body(e_ref, wc_ref, c0_ref, c1_ref):
    dn = (((1,), (1,)), ((), ()))
    ce = lax.dot_general(e_ref[...], wc_ref[...], dn,
                         preferred_element_type=jnp.float32)
    c0_ref[...] = ce[:, :DH]
    c1_ref[...] = ce[:, DH:]


def _edge_mm(e, WC):
    BE = 2000
    grid = (E // BE,)
    return pl.pallas_call(
        _edge_mm_body,
        grid=grid,
        in_specs=[pl.BlockSpec((BE, D), lambda i: (i, 0)),
                  pl.BlockSpec((D, D), lambda i: (0, 0))],
        out_specs=[pl.BlockSpec((BE, DH), lambda i: (i, 0)),
                   pl.BlockSpec((BE, DH), lambda i: (i, 0))],
        out_shape=[jax.ShapeDtypeStruct((E, DH), jnp.float32)] * 2,
    )(e, WC)


def _make_sc_edge_body(p):
    eh_off = p * DH

    def _sc_edge_body(src_hbm, dst_hbm, ce_hbm, db_hbm, ehf_hbm, zeros_hbm,
                      enew_hbm, acc_hbm,
                      idx_s, idx_d, ce_v, db_v, eh_v, nd_v, acc_sh, sem):
        c = lax.axis_index("c")
        s = lax.axis_index("s")
        wid = s * NC + c
        ebase = wid * EPW

        pltpu.sync_copy(zeros_hbm.at[pl.ds(s * RPT, RPT)],
                        acc_sh.at[pl.ds(s * RPT, RPT)])
        plsc.subcore_barrier()

        def chunk_body(ci, carry):
            base = ebase + ci * CHUNK
            pltpu.sync_copy(src_hbm.at[pl.ds(base, CHUNK)], idx_s)
            pltpu.sync_copy(dst_hbm.at[pl.ds(base, CHUNK)], idx_d)
            pltpu.async_copy(db_hbm.at[idx_s], db_v, sem).wait()
            pltpu.async_copy(ehf_hbm.at[idx_d], eh_v, sem).wait()
            pltpu.sync_copy(ce_hbm.at[pl.ds(base, CHUNK)], ce_v)

            def edge_body(i, carry2):
                for j in range(DH // 16):
                    sl = pl.ds(j * 16, 16)
                    en = (ce_v[i, sl] + db_v[i, sl]
                          + eh_v[i, pl.ds(eh_off + j * 16, 16)])
                    ce_v[i, sl] = en
                    sig = 1.0 / (1.0 + jnp.exp(-en))
                    nd_v[i, pl.ds(DH + j * 16, 16)] = sig
                    nd_v[i, pl.ds(j * 16, 16)] = sig * db_v[i, pl.ds(DH + j * 16, 16)]
                return carry2

            lax.fori_loop(0, CHUNK, edge_body, 0)
            pltpu.sync_copy(ce_v, enew_hbm.at[pl.ds(base, CHUNK)])
            pltpu.sync_copy(nd_v, acc_sh.at[idx_d], add=True)
            return carry

        lax.fori_loop(0, NCHUNK, chunk_body, 0)
        plsc.subcore_barrier()
        pltpu.sync_copy(acc_sh.at[pl.ds(s * RPT, RPT)],
                        acc_hbm.at[c, pl.ds(s * RPT, RPT)])

    return _sc_edge_body


def _make_sc_pass(p):
    return pl.kernel(
        _make_sc_edge_body(p),
        out_type=[jax.ShapeDtypeStruct((E, DH), jnp.float32),
                  jax.ShapeDtypeStruct((NC, NPAD, 2 * DH), jnp.float32)],
        mesh=plsc.VectorSubcoreMesh(core_axis_name="c", subcore_axis_name="s"),
        scratch_types=[
            pltpu.VMEM((CHUNK,), jnp.int32),
            pltpu.VMEM((CHUNK,), jnp.int32),
            pltpu.VMEM((CHUNK, DH), jnp.float32),
            pltpu.VMEM((CHUNK, D), jnp.float32),
            pltpu.VMEM((CHUNK, D), jnp.float32),
            pltpu.VMEM((CHUNK, 2 * DH), jnp.float32),
            pltpu.VMEM_SHARED((NPAD, 2 * DH), jnp.float32),
            pltpu.SemaphoreType.DMA,
        ],
    )


_sc_pass0 = _make_sc_pass(0)
_sc_pass1 = _make_sc_pass(1)


def _node_fin_body(h_ref, ah_ref, acc0_ref, acc1_ref, sn_ref, g_ref, b_ref,
                   out_ref, stat_ref):
    j = pl.program_id(0)
    a0 = acc0_ref[0] + acc0_ref[1]
    a1 = acc1_ref[0] + acc1_ref[1]
    num = jnp.concatenate([a0[:, :DH], a1[:, :DH]], axis=1)
    den = jnp.concatenate([a0[:, DH:], a1[:, DH:]], axis=1)
    hn = (ah_ref[...] + num / (den + AGG_EPS)) * sn_ref[...]

    @pl.when(jnp.logical_and(j == 0, pl.program_id(1) == 0))
    def _():
        stat_ref[...] = jnp.zeros_like(stat_ref)

    @pl.when(j == 0)
    def _():
        stat_ref[0:1, :] += jnp.sum(hn, axis=0, keepdims=True)
        stat_ref[1:2, :] += jnp.sum(hn * hn, axis=0, keepdims=True)

    @pl.when(j == 1)
    def _():
        m = stat_ref[0:1, :] / N
        v = stat_ref[1:2, :] / N - m * m
        scale = g_ref[...] / jnp.sqrt(v + BN_EPS)
        out_ref[...] = h_ref[...] + (hn - m) * scale + b_ref[...]


def _node_fin(h, ah, acc0, acc1, snorm_n, gamma_h, beta_h):
    BN = 2000
    grid = (2, N // BN)
    full = pl.BlockSpec((BN, D), lambda j, i: (i, 0))
    accs = pl.BlockSpec((NC, BN, D), lambda j, i: (0, i, 0))
    vec = pl.BlockSpec((1, D), lambda j, i: (0, 0))
    return pl.pallas_call(
        _node_fin_body,
        grid=grid,
        in_specs=[full, full, accs, accs,
                  pl.BlockSpec((BN, 1), lambda j, i: (i, 0)), vec, vec],
        out_specs=full,
        out_shape=jax.ShapeDtypeStruct((N, D), jnp.float32),
        scratch_shapes=[pltpu.VMEM((8, D), jnp.float32)],
    )(h, ah, acc0, acc1, snorm_n, gamma_h, beta_h)


def _edge_fin_body(e_ref, en0_ref, en1_ref, sn_ref, g_ref, b_ref,
                   out_ref, stat_ref):
    j = pl.program_id(0)
    y = jnp.concatenate([en0_ref[...], en1_ref[...]], axis=1) * sn_ref[...]

    @pl.when(jnp.logical_and(j == 0, pl.program_id(1) == 0))
    def _():
        stat_ref[...] = jnp.zeros_like(stat_ref)

    @pl.when(j == 0)
    def _():
        stat_ref[0:1, :] += jnp.sum(y, axis=0, keepdims=True)
        stat_ref[1:2, :] += jnp.sum(y * y, axis=0, keepdims=True)

    @pl.when(j == 1)
    def _():
        m = stat_ref[0:1, :] / E
        v = stat_ref[1:2, :] / E - m * m
        scale = g_ref[...] / jnp.sqrt(v + BN_EPS)
        out_ref[...] = e_ref[...] + (y - m) * scale + b_ref[...]


def _edge_fin(e, en0, en1, snorm_e, gamma_e, beta_e):
    BE = 2000
    grid = (2, E // BE)
    half = pl.BlockSpec((BE, DH), lambda j, i: (i, 0))
    vec = pl.BlockSpec((1, D), lambda j, i: (0, 0))
    lazy = pl.BlockSpec((BE, D), lambda j, i: (jnp.where(j == 1, i, 0), 0))
    return pl.pallas_call(
        _edge_fin_body,
        grid=grid,
        in_specs=[lazy, half, half,
                  pl.BlockSpec((BE, 1), lambda j, i: (i, 0)), vec, vec],
        out_specs=lazy,
        out_shape=jax.ShapeDtypeStruct((E, D), jnp.float32),
        scratch_shapes=[pltpu.VMEM((8, D), jnp.float32)],
    )(e, en0, en1, snorm_e, gamma_e, beta_e)


def kernel(h, e, edge_index, snorm_n, snorm_e, WA, WB, WC, WD, WE,
           gamma_h, beta_h, gamma_e, beta_e):
    src = edge_index[0]
    dst = edge_index[1]
    ah, db0, db1, ehf = _node_mm(h, WA, WB, WD, WE)
    ce0, ce1 = _edge_mm(e, WC)
    zeros = jnp.zeros((NPAD, 2 * DH), jnp.float32)
    en0, acc0 = _sc_pass0(src, dst, ce0, db0, ehf, zeros)
    en1, acc1 = _sc_pass1(src, dst, ce1, db1, ehf, zeros)
    h_out = _node_fin(h, ah, acc0, acc1, snorm_n,
                      gamma_h.reshape(1, D), beta_h.reshape(1, D))
    e_out = _edge_fin(e, en0, en1, snorm_e,
                      gamma_e.reshape(1, D), beta_e.reshape(1, D))
    return (h_out, e_out)


# speedup vs baseline: 2.8168x; 1.4502x over previous
import functools

import jax
import jax.numpy as jnp
from jax import lax
from jax.experimental import pallas as pl
from jax.experimental.pallas import tpu as pltpu
from jax.experimental.pallas import tpu_sc as plsc

N = 10000
E = 320000
D = 128
DH = 64

NC = 2
NS = 16
NW = NC * NS
EPW = E // NW
CHUNK = 40
NCHUNK = EPW // CHUNK
NPAD = 10112
RPT = NPAD // NS

BN_EPS = 1e-5
AGG_EPS = 1e-6


def _node_mm_body(h_ref, wa_ref, wb_ref, wd_ref, we_ref,
                  ah_ref, db0_ref, db1_ref, ehf_ref):
    h = h_ref[...]
    dn = (((1,), (1,)), ((), ()))
    ah_ref[...] = lax.dot_general(h, wa_ref[...], dn,
                                  preferred_element_type=jnp.float32)
    bh = lax.dot_general(h, wb_ref[...], dn, preferred_element_type=jnp.float32)
    dh = lax.dot_general(h, wd_ref[...], dn, preferred_element_type=jnp.float32)
    ehf_ref[...] = lax.dot_general(h, we_ref[...], dn,
                                   preferred_element_type=jnp.float32)
    db0_ref[...] = jnp.concatenate([dh[:, :DH], bh[:, :DH]], axis=1)
    db1_ref[...] = jnp.concatenate([dh[:, DH:], bh[:, DH:]], axis=1)


def _node_mm(h, WA, WB, WD, WE):
    BN = 1000
    grid = (N // BN,)
    full = pl.BlockSpec((BN, D), lambda i: (i, 0))
    w = pl.BlockSpec((D, D), lambda i: (0, 0))
    return pl.pallas_call(
        _node_mm_body,
        grid=grid,
        in_specs=[full, w, w, w, w],
        out_specs=[full, full, full, full],
        out_shape=[jax.ShapeDtypeStruct((N, D), jnp.float32)] * 4,
    )(h, WA, WB, WD, WE)


def _edge_mm_body(e_ref, wc_ref, c0_ref, c1_ref):
    dn = (((1,), (1,)), ((), ()))
    ce = lax.dot_general(e_ref[...], wc_ref[...], dn,
                         preferred_element_type=jnp.float32)
    c0_ref[...] = ce[:, :DH]
    c1_ref[...] = ce[:, DH:]


def _edge_mm(e, WC):
    BE = 2000
    grid = (E // BE,)
    return pl.pallas_call(
        _edge_mm_body,
        grid=grid,
        in_specs=[pl.BlockSpec((BE, D), lambda i: (i, 0)),
                  pl.BlockSpec((D, D), lambda i: (0, 0))],
        out_specs=[pl.BlockSpec((BE, DH), lambda i: (i, 0)),
                   pl.BlockSpec((BE, DH), lambda i: (i, 0))],
        out_shape=[jax.ShapeDtypeStruct((E, DH), jnp.float32)] * 2,
    )(e, WC)


def _make_sc_edge_body(p):
    eh_off = p * DH

    def _sc_edge_body(src_hbm, dst_hbm, ce_hbm, db_hbm, ehf_hbm, zeros_hbm,
                      enew_hbm, acc_hbm,
                      idx_s0, idx_d0, sidx0, ce0, db0, eh0, nd0,
                      idx_s1, idx_d1, sidx1, ce1, db1, eh1, nd1,
                      en_v, acc_sh,
                      isem0, gsem0, osem0, isem1, gsem1, osem1, esem):
        c = lax.axis_index("c")
        s = lax.axis_index("s")
        wid = s * NC + c
        ebase = wid * EPW
        bufs = ((idx_s0, idx_d0, sidx0, ce0, db0, eh0, nd0,
                 isem0, gsem0, osem0),
                (idx_s1, idx_d1, sidx1, ce1, db1, eh1, nd1,
                 isem1, gsem1, osem1))

        pltpu.sync_copy(zeros_hbm.at[pl.ds(s * RPT, RPT)],
                        acc_sh.at[pl.ds(s * RPT, RPT)])
        plsc.subcore_barrier()

        def issue_inputs(ci, b):
            idx_s, idx_d, _, ce_v, _, _, _, isem, _, _ = bufs[b]
            base = ebase + ci * CHUNK
            pltpu.async_copy(src_hbm.at[pl.ds(base, CHUNK)], idx_s, isem)
            pltpu.async_copy(dst_hbm.at[pl.ds(base, CHUNK)], idx_d, isem)
            pltpu.async_copy(ce_hbm.at[pl.ds(base, CHUNK)], ce_v, isem)

        def launch_gathers(b):
            idx_s, idx_d, _, ce_v, db_v, eh_v, _, isem, gsem, _ = bufs[b]
            pltpu.make_async_copy(src_hbm.at[pl.ds(0, CHUNK)], idx_s,
                                  isem).wait()
            pltpu.make_async_copy(dst_hbm.at[pl.ds(0, CHUNK)], idx_d,
                                  isem).wait()
            pltpu.make_async_copy(ce_hbm.at[pl.ds(0, CHUNK)], ce_v,
                                  isem).wait()
            pltpu.async_copy(db_hbm.at[idx_s], db_v, gsem)
            pltpu.async_copy(ehf_hbm.at[idx_d], eh_v, gsem)

        def handle(ci, b):
            idx_s, idx_d, sidx, ce_v, db_v, eh_v, nd_v, isem, gsem, \
                osem = bufs[b]

            @pl.when(ci + 1 < NCHUNK)
            def _():
                launch_gathers(1 - b)

            pltpu.make_async_copy(db_hbm.at[idx_s], db_v, gsem).wait()
            pltpu.make_async_copy(ehf_hbm.at[idx_d], eh_v, gsem).wait()

            @pl.when(ci >= 1)
            def _():
                pltpu.make_async_copy(en_v, enew_hbm.at[pl.ds(0, CHUNK)],
                                      esem).wait()

            @pl.when(ci >= 2)
            def _():
                pltpu.make_async_copy(nd_v, acc_sh.at[sidx], osem).wait()

            def edge_body(i, carry2):
                for j in range(DH // 16):
                    sl = pl.ds(j * 16, 16)
                    en = (ce_v[i, sl] + db_v[i, sl]
                          + eh_v[i, pl.ds(eh_off + j * 16, 16)])
                    en_v[i, sl] = en
                    sig = 1.0 / (1.0 + jnp.exp(-en))
                    nd_v[i, pl.ds(DH + j * 16, 16)] = sig
                    nd_v[i, pl.ds(j * 16, 16)] = (
                        sig * db_v[i, pl.ds(DH + j * 16, 16)])
                return carry2

            lax.fori_loop(0, CHUNK, edge_body, 0)
            offs = sorted({min(k * 16, CHUNK - 16)
                           for k in range((CHUNK + 15) // 16)})
            for o in offs:
                sidx[pl.ds(o, 16)] = idx_d[pl.ds(o, 16)]
            base = ebase + ci * CHUNK
            pltpu.async_copy(en_v, enew_hbm.at[pl.ds(base, CHUNK)], esem)
            pltpu.async_copy(nd_v, acc_sh.at[sidx], osem, add=True)

            @pl.when(ci + 2 < NCHUNK)
            def _():
                issue_inputs(ci + 2, b)

        issue_inputs(0, 0)
        issue_inputs(1, 1)
        launch_gathers(0)

        def pair_body(g, carry):
            handle(2 * g, 0)
            handle(2 * g + 1, 1)
            return carry

        lax.fori_loop(0, NCHUNK // 2, pair_body, 0)

        pltpu.make_async_copy(en_v, enew_hbm.at[pl.ds(0, CHUNK)], esem).wait()
        for b in (0, 1):
            _, _, sidx, _, _, _, nd_v, _, _, osem = bufs[b]
            pltpu.make_async_copy(nd_v, acc_sh.at[sidx], osem).wait()

        plsc.subcore_barrier()
        pltpu.sync_copy(acc_sh.at[pl.ds(s * RPT, RPT)],
                        acc_hbm.at[c, pl.ds(s * RPT, RPT)])

    return _sc_edge_body


def _make_sc_pass(p):
    buf = [
        pltpu.VMEM((CHUNK,), jnp.int32),
        pltpu.VMEM((CHUNK,), jnp.int32),
        pltpu.VMEM((CHUNK,), jnp.int32),
        pltpu.VMEM((CHUNK, DH), jnp.float32),
        pltpu.VMEM((CHUNK, D), jnp.float32),
        pltpu.VMEM((CHUNK, D), jnp.float32),
        pltpu.VMEM((CHUNK, 2 * DH), jnp.float32),
    ]
    return pl.kernel(
        _make_sc_edge_body(p),
        out_type=[jax.ShapeDtypeStruct((E, DH), jnp.float32),
                  jax.ShapeDtypeStruct((NC, NPAD, 2 * DH), jnp.float32)],
        mesh=plsc.VectorSubcoreMesh(core_axis_name="c", subcore_axis_name="s"),
        scratch_types=buf + buf + [
            pltpu.VMEM((CHUNK, DH), jnp.float32),
            pltpu.VMEM_SHARED((NPAD, 2 * DH), jnp.float32),
            pltpu.SemaphoreType.DMA, pltpu.SemaphoreType.DMA,
            pltpu.SemaphoreType.DMA, pltpu.SemaphoreType.DMA,
            pltpu.SemaphoreType.DMA, pltpu.SemaphoreType.DMA,
            pltpu.SemaphoreType.DMA,
        ],
    )


_sc_pass0 = _make_sc_pass(0)
_sc_pass1 = _make_sc_pass(1)


def _node_fin_body(h_ref, ah_ref, acc0_ref, acc1_ref, sn_ref, g_ref, b_ref,
                   out_ref, stat_ref):
    j = pl.program_id(0)
    a0 = acc0_ref[0] + acc0_ref[1]
    a1 = acc1_ref[0] + acc1_ref[1]
    num = jnp.concatenate([a0[:, :DH], a1[:, :DH]], axis=1)
    den = jnp.concatenate([a0[:, DH:], a1[:, DH:]], axis=1)
    hn = (ah_ref[...] + num / (den + AGG_EPS)) * sn_ref[...]

    @pl.when(jnp.logical_and(j == 0, pl.program_id(1) == 0))
    def _():
        stat_ref[...] = jnp.zeros_like(stat_ref)

    @pl.when(j == 0)
    def _():
        stat_ref[0:1, :] += jnp.sum(hn, axis=0, keepdims=True)
        stat_ref[1:2, :] += jnp.sum(hn * hn, axis=0, keepdims=True)

    @pl.when(j == 1)
    def _():
        m = stat_ref[0:1, :] / N
        v = stat_ref[1:2, :] / N - m * m
        scale = g_ref[...] / jnp.sqrt(v + BN_EPS)
        out_ref[...] = h_ref[...] + (hn - m) * scale + b_ref[...]


def _node_fin(h, ah, acc0, acc1, snorm_n, gamma_h, beta_h):
    BN = 2000
    grid = (2, N // BN)
    full = pl.BlockSpec((BN, D), lambda j, i: (i, 0))
    accs = pl.BlockSpec((NC, BN, D), lambda j, i: (0, i, 0))
    vec = pl.BlockSpec((1, D), lambda j, i: (0, 0))
    return pl.pallas_call(
        _node_fin_body,
        grid=grid,
        in_specs=[full, full, accs, accs,
                  pl.BlockSpec((BN, 1), lambda j, i: (i, 0)), vec, vec],
        out_specs=full,
        out_shape=jax.ShapeDtypeStruct((N, D), jnp.float32),
        scratch_shapes=[pltpu.VMEM((8, D), jnp.float32)],
    )(h, ah, acc0, acc1, snorm_n, gamma_h, beta_h)


def _edge_fin_body(e_ref, en0_ref, en1_ref, sn_ref, g_ref, b_ref,
                   out_ref, stat_ref):
    j = pl.program_id(0)
    y = jnp.concatenate([en0_ref[...], en1_ref[...]], axis=1) * sn_ref[...]

    @pl.when(jnp.logical_and(j == 0, pl.program_id(1) == 0))
    def _():
        stat_ref[...] = jnp.zeros_like(stat_ref)

    @pl.when(j == 0)
    def _():
        stat_ref[0:1, :] += jnp.sum(y, axis=0, keepdims=True)
        stat_ref[1:2, :] += jnp.sum(y * y, axis=0, keepdims=True)

    @pl.when(j == 1)
    def _():
        m = stat_ref[0:1, :] / E
        v = stat_ref[1:2, :] / E - m * m
        scale = g_ref[...] / jnp.sqrt(v + BN_EPS)
        out_ref[...] = e_ref[...] + (y - m) * scale + b_ref[...]


def _edge_fin(e, en0, en1, snorm_e, gamma_e, beta_e):
    BE = 2000
    grid = (2, E // BE)
    half = pl.BlockSpec((BE, DH), lambda j, i: (i, 0))
    vec = pl.BlockSpec((1, D), lambda j, i: (0, 0))
    lazy = pl.BlockSpec((BE, D), lambda j, i: (jnp.where(j == 1, i, 0), 0))
    return pl.pallas_call(
        _edge_fin_body,
        grid=grid,
        in_specs=[lazy, half, half,
                  pl.BlockSpec((BE, 1), lambda j, i: (i, 0)), vec, vec],
        out_specs=lazy,
        out_shape=jax.ShapeDtypeStruct((E, D), jnp.float32),
        scratch_shapes=[pltpu.VMEM((8, D), jnp.float32)],
    )(e, en0, en1, snorm_e, gamma_e, beta_e)


def kernel(h, e, edge_index, snorm_n, snorm_e, WA, WB, WC, WD, WE,
           gamma_h, beta_h, gamma_e, beta_e):
    src = edge_index[0]
    dst = edge_index[1]
    ah, db0, db1, ehf = _node_mm(h, WA, WB, WD, WE)
    ce0, ce1 = _edge_mm(e, WC)
    zeros = jnp.zeros((NPAD, 2 * DH), jnp.float32)
    en0, acc0 = _sc_pass0(src, dst, ce0, db0, ehf, zeros)
    en1, acc1 = _sc_pass1(src, dst, ce1, db1, ehf, zeros)
    h_out = _node_fin(h, ah, acc0, acc1, snorm_n,
                      gamma_h.reshape(1, D), beta_h.reshape(1, D))
    e_out = _edge_fin(e, en0, en1, snorm_e,
                      gamma_e.reshape(1, D), beta_e.reshape(1, D))
    return (h_out, e_out)
